# Initial kernel scaffold; baseline (speedup 1.0000x reference)
import jax
import jax.numpy as jnp
from jax.experimental import pallas as pl


def kernel(z, pos, batch, embed, Wpos, W1, b1, Wout):
    raise NotImplementedError("write your pallas kernel here")


# speedup vs baseline: 2.0835x; 2.0835x over previous
import functools

import jax
import jax.numpy as jnp
from jax import lax
from jax.experimental import pallas as pl
from jax.experimental.pallas import tpu as pltpu

_N = 100000
_H = 128
_ZP = 128
_G = 1024
_B = 2000


def _tc_body(z_ref, pos_ref, batch_ref, embed_ref, wpos_ref, w1_ref, b1_ref,
             wout_ref, out_ref):
    i = pl.program_id(0)

    z = z_ref[...]
    onehot = (z == lax.broadcasted_iota(jnp.int32, (_B, _ZP), 1)
              ).astype(jnp.float32)
    h = jnp.dot(onehot, embed_ref[...],
                preferred_element_type=jnp.float32)
    h = h + jnp.dot(pos_ref[...], wpos_ref[...],
                    preferred_element_type=jnp.float32)
    x = jnp.dot(h, w1_ref[...],
                preferred_element_type=jnp.float32) + b1_ref[...]
    x = x * jax.nn.sigmoid(x)
    y = jnp.dot(x, wout_ref[...],
                preferred_element_type=jnp.float32)

    seg = (batch_ref[...] == lax.broadcasted_iota(jnp.int32, (_B, _G), 1))
    part = jnp.sum(jnp.where(seg, y, 0.0), axis=0, keepdims=True)

    @pl.when(i == 0)
    def _():
        out_ref[...] = jnp.zeros_like(out_ref)

    out_ref[...] += part


@jax.jit
def kernel(z, pos, batch, embed, Wpos, W1, b1, Wout):
    grid = _N // _B
    embed_p = jnp.zeros((_ZP, _H), jnp.float32).at[:embed.shape[0]].set(embed)
    out = pl.pallas_call(
        _tc_body,
        grid=(grid,),
        in_specs=[
            pl.BlockSpec((_B, 1), lambda i: (i, 0)),
            pl.BlockSpec((_B, 3), lambda i: (i, 0)),
            pl.BlockSpec((_B, 1), lambda i: (i, 0)),
            pl.BlockSpec((_ZP, _H), lambda i: (0, 0)),
            pl.BlockSpec((3, _H), lambda i: (0, 0)),
            pl.BlockSpec((_H, _H), lambda i: (0, 0)),
            pl.BlockSpec((1, _H), lambda i: (0, 0)),
            pl.BlockSpec((_H, 1), lambda i: (0, 0)),
        ],
        out_specs=pl.BlockSpec((1, _G), lambda i: (0, 0)),
        out_shape=jax.ShapeDtypeStruct((1, _G), jnp.float32),
        compiler_params=pltpu.CompilerParams(
            dimension_semantics=("arbitrary",)),
    )(z.reshape(_N, 1), pos, batch.reshape(_N, 1), embed_p, Wpos, W1,
      b1.reshape(1, _H), Wout)
    return out.reshape(_G, 1)
